# baseline (speedup 1.0000x reference)
import functools

import jax
import jax.numpy as jnp
from jax import lax
from jax.experimental import pallas as pl
from jax.experimental.pallas import tpu as pltpu, tpu_sc as plsc

NC = 2
NS = 16
L = 16
CH = 2048

_SC_PARAMS = pltpu.CompilerParams(use_tc_tiling_on_sc=False,
                                  needs_layout_passes=False)


def _mm_body(a_ref, b_ref, o_ref):
    o_ref[...] = jnp.dot(a_ref[...], b_ref[...],
                         preferred_element_type=jnp.float32)


def _mm(a, b):
    m, _ = a.shape
    n = b.shape[1]
    return pl.pallas_call(
        _mm_body,
        out_shape=jax.ShapeDtypeStruct((m, n), jnp.float32),
    )(a, b)


def _mm_split_body(n1, a_ref, b_ref, o1_ref, o2_ref):
    o = jnp.dot(a_ref[...], b_ref[...], preferred_element_type=jnp.float32)
    o1_ref[...] = o[:, :n1]
    o2_ref[...] = o[:, n1:]


def _mm_split(a, b, n1, bm):
    m, k = a.shape
    n = b.shape[1]
    grid = (m // bm,)
    return pl.pallas_call(
        functools.partial(_mm_split_body, n1),
        grid=grid,
        in_specs=[
            pl.BlockSpec((bm, k), lambda i: (i, 0)),
            pl.BlockSpec((k, n), lambda i: (0, 0)),
        ],
        out_specs=[
            pl.BlockSpec((bm, n1), lambda i: (i, 0)),
            pl.BlockSpec((bm, n - n1), lambda i: (i, 0)),
        ],
        out_shape=[
            jax.ShapeDtypeStruct((m, n1), jnp.float32),
            jax.ShapeDtypeStruct((m, n - n1), jnp.float32),
        ],
    )(a, b)


def _relu_mm_split_body(n1, p0_ref, p1_ref, xr_ref, bias_ref, b_ref,
                        o1_ref, o2_ref):
    h = jnp.maximum(p0_ref[...] + p1_ref[...] + xr_ref[...]
                    + bias_ref[0, :], 0.0)
    o = jnp.dot(h, b_ref[...], preferred_element_type=jnp.float32)
    o1_ref[...] = o[:, :n1]
    o2_ref[...] = o[:, n1:]


def _relu_mm_split(p0, p1, xr, bias, b, n1, bm):
    m, k = xr.shape
    n = b.shape[1]
    grid = (m // bm,)
    return pl.pallas_call(
        functools.partial(_relu_mm_split_body, n1),
        grid=grid,
        in_specs=[
            pl.BlockSpec((bm, k), lambda i: (i, 0)),
            pl.BlockSpec((bm, k), lambda i: (i, 0)),
            pl.BlockSpec((bm, k), lambda i: (i, 0)),
            pl.BlockSpec((1, k), lambda i: (0, 0)),
            pl.BlockSpec((k, n), lambda i: (0, 0)),
        ],
        out_specs=[
            pl.BlockSpec((bm, n1), lambda i: (i, 0)),
            pl.BlockSpec((bm, n - n1), lambda i: (i, 0)),
        ],
        out_shape=[
            jax.ShapeDtypeStruct((m, n1), jnp.float32),
            jax.ShapeDtypeStruct((m, n - n1), jnp.float32),
        ],
    )(p0, p1, xr, bias, b)


def _logsoftmax_body(p0_ref, p1_ref, xr_ref, bias_ref, o_ref):
    z = p0_ref[...] + p1_ref[...] + xr_ref[...] + bias_ref[0, :]
    m = jnp.max(z, axis=1, keepdims=True)
    lse = jnp.log(jnp.sum(jnp.exp(z - m), axis=1, keepdims=True)) + m
    o_ref[...] = z - lse


def _final_logsoftmax(p0, p1, xr, bias, bm):
    m, k = xr.shape
    grid = (m // bm,)
    spec = pl.BlockSpec((bm, k), lambda i: (i, 0))
    return pl.pallas_call(
        _logsoftmax_body,
        grid=grid,
        in_specs=[spec, spec, spec, pl.BlockSpec((1, k), lambda i: (0, 0))],
        out_specs=spec,
        out_shape=jax.ShapeDtypeStruct((m, k), jnp.float32),
    )(p0, p1, xr, bias)


def _zero_rows(buf_ref, n):
    def body(i, _):
        buf_ref[i] = jnp.zeros((L,), jnp.float32)
        return 0
    lax.fori_loop(0, n, body, 0)


def _zero_flat(buf_ref, n):
    def body(i, _):
        buf_ref[pl.ds(i * L, L)] = jnp.zeros((L,), jnp.float32)
        return 0
    lax.fori_loop(0, n // L, body, 0)


def _scale_rows(rows_ref, norm_ref, n):
    def body(jj, _):
        for u in range(8):
            j = jj * 8 + u
            nb = plsc.load_gather(norm_ref, [jnp.full((L,), j, jnp.int32)])
            rows_ref[j] = rows_ref[j] * nb
        return 0
    lax.fori_loop(0, n // 8, body, 0)


def _fill_groups(ref, first_group, value):
    def body(g, _):
        ref[pl.ds(g * L, L)] = jnp.full((L,), value, jnp.int32)
        return 0
    lax.fori_loop(first_group, CH // L, body, 0)


def _make_sc_layer1(n_nodes, n_rel, e):
    assert e % (NC * NS * 8) == 0
    qt1 = e // NS
    qt3 = e // (NC * NS)

    deg_flat = -(-((n_nodes + 1) * n_rel) // (NS * L)) * (NS * L)
    dpt = deg_flat // NS
    trash_deg = n_nodes * n_rel
    out_rows = -(-(n_nodes + 1) // (NS * 8)) * (NS * 8)
    opt = out_rows // NS
    trash_row = n_nodes

    mesh = plsc.VectorSubcoreMesh(core_axis_name="c", subcore_axis_name="s",
                                  num_cores=NC, num_subcores=NS)

    @functools.partial(
        pl.kernel, mesh=mesh,
        compiler_params=_SC_PARAMS,
        out_type=(
            jax.ShapeDtypeStruct((out_rows, L), jnp.float32),
            jax.ShapeDtypeStruct((out_rows, L), jnp.float32),
            jax.ShapeDtypeStruct((e,), jnp.float32),
            jax.ShapeDtypeStruct((e,), jnp.int32),
            jax.ShapeDtypeStruct((deg_flat,), jnp.float32),
            jax.ShapeDtypeStruct((deg_flat,), jnp.float32),
        ),
        scratch_types=[
            pltpu.VMEM_SHARED((deg_flat,), jnp.float32),
            pltpu.VMEM_SHARED((out_rows, L), jnp.float32),
            pltpu.VMEM((CH,), jnp.int32),
            pltpu.VMEM((CH,), jnp.int32),
            pltpu.VMEM((CH,), jnp.int32),
            pltpu.VMEM((CH,), jnp.int32),
            pltpu.VMEM((CH,), jnp.int32),
            pltpu.VMEM((CH,), jnp.float32),
            pltpu.VMEM((CH,), jnp.float32),
            pltpu.VMEM((CH,), jnp.float32),
            pltpu.VMEM((CH, L), jnp.float32),
            pltpu.VMEM((opt, L), jnp.float32),
            pltpu.VMEM((dpt,), jnp.float32),
            pltpu.SemaphoreType.DMA,
        ],
    )
    def k(ei, tp, ytab, outp0, outp1, normp, idxgp, deg0, deg1,
          sdeg, sout, src_v, dst_v, t_v, idxd_v, idxg_v, ones_v, dval_v,
          norm_v, yrows_v, stage_v, zbuf_v, sem):
        c = lax.axis_index("c")
        s = lax.axis_index("s")

        def init_ones(i, _):
            ones_v[pl.ds(i * L, L)] = jnp.full((L,), 1.0, jnp.float32)
            return 0
        lax.fori_loop(0, CH // L, init_ones, 0)
        _zero_flat(zbuf_v, dpt)
        _zero_rows(stage_v, opt)
        pltpu.sync_copy(zbuf_v, sdeg.at[pl.ds(s * dpt, dpt)])
        pltpu.sync_copy(stage_v, sout.at[pl.ds(s * opt, opt)])
        plsc.subcore_barrier()

        def p1_chunk(eb, n_real):
            pltpu.sync_copy(ei.at[1, pl.ds(eb, n_real)],
                            dst_v.at[pl.ds(0, n_real)])
            pltpu.sync_copy(tp.at[pl.ds(eb, n_real)],
                            t_v.at[pl.ds(0, n_real)])
            fg = n_real // L

            def mk_idxd(g, _):
                sl = pl.ds(g * L, L)
                idxd_v[sl] = dst_v[sl] * n_rel + t_v[sl]
                return 0
            lax.fori_loop(0, fg, mk_idxd, 0)
            if n_real % L:
                sl = pl.ds(fg * L, L)
                m = lax.iota(jnp.int32, L) < (n_real % L)
                idxd_v[sl] = jnp.where(m, dst_v[sl] * n_rel + t_v[sl],
                                       trash_deg)
                fg += 1
            _fill_groups(idxd_v, fg, trash_deg)
            pltpu.sync_copy(ones_v, sdeg.at[idxd_v], add=True)

        nfull1 = qt1 // CH

        def p1_loop(kk, _):
            p1_chunk(s * qt1 + kk * CH, CH)
            return 0
        lax.fori_loop(0, nfull1, p1_loop, 0)
        if qt1 % CH:
            p1_chunk(s * qt1 + nfull1 * CH, qt1 % CH)
        plsc.subcore_barrier()

        def deg_out(degref):
            pltpu.sync_copy(sdeg.at[pl.ds(s * dpt, dpt)], zbuf_v)
            pltpu.sync_copy(zbuf_v, degref.at[pl.ds(s * dpt, dpt)])

        @pl.when(c == 0)
        def _():
            deg_out(deg0)

        @pl.when(c == 1)
        def _():
            deg_out(deg1)
        plsc.subcore_barrier()

        def p3_chunk(eb, n_real):
            pltpu.sync_copy(ei.at[0, pl.ds(eb, n_real)],
                            src_v.at[pl.ds(0, n_real)])
            pltpu.sync_copy(ei.at[1, pl.ds(eb, n_real)],
                            dst_v.at[pl.ds(0, n_real)])
            pltpu.sync_copy(tp.at[pl.ds(eb, n_real)],
                            t_v.at[pl.ds(0, n_real)])
            fg = n_real // L

            def mk_idx(g, _):
                sl = pl.ds(g * L, L)
                tt = t_v[sl]
                idxg_v[sl] = src_v[sl] * n_rel + tt
                idxd_v[sl] = dst_v[sl] * n_rel + tt
                return 0
            lax.fori_loop(0, fg, mk_idx, 0)
            if n_real % L:
                sl = pl.ds(fg * L, L)
                m = lax.iota(jnp.int32, L) < (n_real % L)
                tt = t_v[sl]
                idxg_v[sl] = jnp.where(m, src_v[sl] * n_rel + tt, 0)
                idxd_v[sl] = jnp.where(m, dst_v[sl] * n_rel + tt, trash_deg)
                dst_v[sl] = jnp.where(m, dst_v[sl], trash_row)
                fg += 1
            _fill_groups(idxg_v, fg, 0)
            _fill_groups(idxd_v, fg, trash_deg)
            _fill_groups(dst_v, fg, trash_row)

            pltpu.sync_copy(idxg_v.at[pl.ds(0, n_real)],
                            idxgp.at[pl.ds(eb, n_real)])

            @pl.when(c == 0)
            def _():
                pltpu.async_copy(deg0.at[idxd_v], dval_v, sem).wait()

            @pl.when(c == 1)
            def _():
                pltpu.async_copy(deg1.at[idxd_v], dval_v, sem).wait()

            def mk_norm(g, _):
                sl = pl.ds(g * L, L)
                norm_v[sl] = 1.0 / jnp.maximum(dval_v[sl], 1.0)
                return 0
            lax.fori_loop(0, CH // L, mk_norm, 0)
            pltpu.sync_copy(norm_v.at[pl.ds(0, n_real)],
                            normp.at[pl.ds(eb, n_real)])

            pltpu.async_copy(ytab.at[idxg_v], yrows_v, sem).wait()
            _scale_rows(yrows_v, norm_v, CH)
            pltpu.sync_copy(yrows_v, sout.at[dst_v], add=True)

        nfull3 = qt3 // CH

        def p3_loop(kk, _):
            p3_chunk(c * (e // NC) + s * qt3 + kk * CH, CH)
            return 0
        lax.fori_loop(0, nfull3, p3_loop, 0)
        if qt3 % CH:
            p3_chunk(c * (e // NC) + s * qt3 + nfull3 * CH, qt3 % CH)
        plsc.subcore_barrier()

        def part_out(oref):
            pltpu.sync_copy(sout.at[pl.ds(s * opt, opt)], stage_v)
            pltpu.sync_copy(stage_v, oref.at[pl.ds(s * opt, opt)])

        @pl.when(c == 0)
        def _():
            part_out(outp0)

        @pl.when(c == 1)
        def _():
            part_out(outp1)

    return k


def _make_sc_layer2(n_nodes, e):
    assert e % (NC * NS * 8) == 0
    qt = e // (NC * NS)
    out_rows = -(-(n_nodes + 1) // (NS * 8)) * (NS * 8)
    opt = out_rows // NS
    trash_row = n_nodes

    mesh = plsc.VectorSubcoreMesh(core_axis_name="c", subcore_axis_name="s",
                                  num_cores=NC, num_subcores=NS)

    @functools.partial(
        pl.kernel, mesh=mesh,
        compiler_params=_SC_PARAMS,
        out_type=(
            jax.ShapeDtypeStruct((out_rows, L), jnp.float32),
            jax.ShapeDtypeStruct((out_rows, L), jnp.float32),
        ),
        scratch_types=[
            pltpu.VMEM_SHARED((out_rows, L), jnp.float32),
            pltpu.VMEM((CH,), jnp.int32),
            pltpu.VMEM((CH,), jnp.int32),
            pltpu.VMEM((CH,), jnp.float32),
            pltpu.VMEM((CH, L), jnp.float32),
            pltpu.VMEM((opt, L), jnp.float32),
            pltpu.SemaphoreType.DMA,
        ],
    )
    def k(idxgp, ei, normp, ytab, outp0, outp1,
          sout, idxg_v, dst_v, norm_v, yrows_v, stage_v, sem):
        c = lax.axis_index("c")
        s = lax.axis_index("s")

        _zero_rows(stage_v, opt)
        pltpu.sync_copy(stage_v, sout.at[pl.ds(s * opt, opt)])
        plsc.subcore_barrier()

        def chunk(eb, n_real):
            pltpu.sync_copy(idxgp.at[pl.ds(eb, n_real)],
                            idxg_v.at[pl.ds(0, n_real)])
            pltpu.sync_copy(ei.at[1, pl.ds(eb, n_real)],
                            dst_v.at[pl.ds(0, n_real)])
            pltpu.sync_copy(normp.at[pl.ds(eb, n_real)],
                            norm_v.at[pl.ds(0, n_real)])
            fg = n_real // L
            if n_real % L:
                sl = pl.ds(fg * L, L)
                m = lax.iota(jnp.int32, L) < (n_real % L)
                idxg_v[sl] = jnp.where(m, idxg_v[sl], 0)
                dst_v[sl] = jnp.where(m, dst_v[sl], trash_row)
                fg += 1
            _fill_groups(idxg_v, fg, 0)
            _fill_groups(dst_v, fg, trash_row)
            pltpu.async_copy(ytab.at[idxg_v], yrows_v, sem).wait()
            _scale_rows(yrows_v, norm_v, CH)
            pltpu.sync_copy(yrows_v, sout.at[dst_v], add=True)

        nfull = qt // CH

        def loop(kk, _):
            chunk(c * (e // NC) + s * qt + kk * CH, CH)
            return 0
        lax.fori_loop(0, nfull, loop, 0)
        if qt % CH:
            chunk(c * (e // NC) + s * qt + nfull * CH, qt % CH)
        plsc.subcore_barrier()

        def part_out(oref):
            pltpu.sync_copy(sout.at[pl.ds(s * opt, opt)], stage_v)
            pltpu.sync_copy(stage_v, oref.at[pl.ds(s * opt, opt)])

        @pl.when(c == 0)
        def _():
            part_out(outp0)

        @pl.when(c == 1)
        def _():
            part_out(outp1)

    return k


def kernel(x, edge_index, edge_type, comp1, bases1, root1, bias1,
           comp2, bases2, root2, bias2):
    n_nodes, in_c = x.shape
    e = edge_index.shape[1]
    n_rel, n_bases = comp1.shape
    hid = bases1.shape[2]
    n_cls = bases2.shape[2]
    assert hid == L and n_cls == L

    w1 = _mm(comp1, bases1.reshape(n_bases, in_c * hid))
    w1 = w1.reshape(n_rel, in_c, hid).transpose(1, 0, 2).reshape(
        in_c, n_rel * hid)
    w2 = _mm(comp2, bases2.reshape(n_bases, hid * n_cls))
    w2 = w2.reshape(n_rel, hid, n_cls).transpose(1, 0, 2).reshape(
        hid, n_rel * n_cls)

    y1, xr1 = _mm_split(x, jnp.concatenate([w1, root1], axis=1),
                        n_rel * hid, 1000)
    y1tab = y1.reshape(n_nodes * n_rel, hid)

    sc1 = _make_sc_layer1(n_nodes, n_rel, e)
    p0, p1, normp, idxgp, _, _ = sc1(edge_index, edge_type, y1tab)

    y2, xr2 = _relu_mm_split(
        p0, p1, xr1, bias1.reshape(1, hid),
        jnp.concatenate([w2, root2], axis=1), n_rel * n_cls, 1000)
    y2tab = y2.reshape(n_nodes * n_rel, n_cls)

    sc2 = _make_sc_layer2(n_nodes, e)
    q0, q1 = sc2(idxgp, edge_index, normp, y2tab)

    return _final_logsoftmax(q0, q1, xr2, bias2.reshape(1, n_cls), 1000)


# speedup vs baseline: 2.2521x; 2.2521x over previous
import functools

import jax
import jax.numpy as jnp
from jax import lax
from jax.experimental import pallas as pl
from jax.experimental.pallas import tpu as pltpu, tpu_sc as plsc

NC = 2
NS = 16
L = 16
CH = 1024

_SC_PARAMS = pltpu.CompilerParams(use_tc_tiling_on_sc=False,
                                  needs_layout_passes=False)


def _mm_body(a_ref, b_ref, o_ref):
    o_ref[...] = jnp.dot(a_ref[...], b_ref[...],
                         preferred_element_type=jnp.float32)


def _mm(a, b):
    m, _ = a.shape
    n = b.shape[1]
    return pl.pallas_call(
        _mm_body,
        out_shape=jax.ShapeDtypeStruct((m, n), jnp.float32),
    )(a, b)


def _mm_split_body(n1, a_ref, b_ref, o1_ref, o2_ref):
    o = jnp.dot(a_ref[...], b_ref[...], preferred_element_type=jnp.float32)
    o1_ref[...] = o[:, :n1]
    o2_ref[...] = o[:, n1:]


def _mm_split(a, b, n1, bm):
    m, k = a.shape
    n = b.shape[1]
    grid = (m // bm,)
    return pl.pallas_call(
        functools.partial(_mm_split_body, n1),
        grid=grid,
        in_specs=[
            pl.BlockSpec((bm, k), lambda i: (i, 0)),
            pl.BlockSpec((k, n), lambda i: (0, 0)),
        ],
        out_specs=[
            pl.BlockSpec((bm, n1), lambda i: (i, 0)),
            pl.BlockSpec((bm, n - n1), lambda i: (i, 0)),
        ],
        out_shape=[
            jax.ShapeDtypeStruct((m, n1), jnp.float32),
            jax.ShapeDtypeStruct((m, n - n1), jnp.float32),
        ],
    )(a, b)


def _relu_mm_split_body(n1, p0_ref, p1_ref, xr_ref, bias_ref, b_ref,
                        o1_ref, o2_ref):
    h = jnp.maximum(p0_ref[...] + p1_ref[...] + xr_ref[...]
                    + bias_ref[0, :], 0.0)
    o = jnp.dot(h, b_ref[...], preferred_element_type=jnp.float32)
    o1_ref[...] = o[:, :n1]
    o2_ref[...] = o[:, n1:]


def _relu_mm_split(p0, p1, xr, bias, b, n1, bm):
    m, k = xr.shape
    n = b.shape[1]
    grid = (m // bm,)
    return pl.pallas_call(
        functools.partial(_relu_mm_split_body, n1),
        grid=grid,
        in_specs=[
            pl.BlockSpec((bm, k), lambda i: (i, 0)),
            pl.BlockSpec((bm, k), lambda i: (i, 0)),
            pl.BlockSpec((bm, k), lambda i: (i, 0)),
            pl.BlockSpec((1, k), lambda i: (0, 0)),
            pl.BlockSpec((k, n), lambda i: (0, 0)),
        ],
        out_specs=[
            pl.BlockSpec((bm, n1), lambda i: (i, 0)),
            pl.BlockSpec((bm, n - n1), lambda i: (i, 0)),
        ],
        out_shape=[
            jax.ShapeDtypeStruct((m, n1), jnp.float32),
            jax.ShapeDtypeStruct((m, n - n1), jnp.float32),
        ],
    )(p0, p1, xr, bias, b)


def _logsoftmax_body(p0_ref, p1_ref, xr_ref, bias_ref, o_ref):
    z = p0_ref[...] + p1_ref[...] + xr_ref[...] + bias_ref[0, :]
    m = jnp.max(z, axis=1, keepdims=True)
    lse = jnp.log(jnp.sum(jnp.exp(z - m), axis=1, keepdims=True)) + m
    o_ref[...] = z - lse


def _final_logsoftmax(p0, p1, xr, bias, bm):
    m, k = xr.shape
    grid = (m // bm,)
    spec = pl.BlockSpec((bm, k), lambda i: (i, 0))
    return pl.pallas_call(
        _logsoftmax_body,
        grid=grid,
        in_specs=[spec, spec, spec, pl.BlockSpec((1, k), lambda i: (0, 0))],
        out_specs=spec,
        out_shape=jax.ShapeDtypeStruct((m, k), jnp.float32),
    )(p0, p1, xr, bias)


def _zero_rows(buf_ref, n):
    def body(i, _):
        buf_ref[i] = jnp.zeros((L,), jnp.float32)
        return 0
    lax.fori_loop(0, n, body, 0)


def _zero_flat(buf_ref, n):
    def body(i, _):
        buf_ref[pl.ds(i * L, L)] = jnp.zeros((L,), jnp.float32)
        return 0
    lax.fori_loop(0, n // L, body, 0)


def _scale_rows(rows_ref, norm_ref, n):
    def body(jj, _):
        for u in range(8):
            j = jj * 8 + u
            nb = plsc.load_gather(norm_ref, [jnp.full((L,), j, jnp.int32)])
            rows_ref[j] = rows_ref[j] * nb
        return 0
    lax.fori_loop(0, n // 8, body, 0)


def _fill_groups(ref, first_group, value):
    def body(g, _):
        ref[pl.ds(g * L, L)] = jnp.full((L,), value, jnp.int32)
        return 0
    lax.fori_loop(first_group, CH // L, body, 0)


def _make_sc_layer1(n_nodes, n_rel, e):
    assert e % (NC * NS * 8) == 0
    qt1 = e // NS
    qt3 = e // (NC * NS)

    deg_flat = -(-((n_nodes + 1) * n_rel) // (NS * L)) * (NS * L)
    dpt = deg_flat // NS
    trash_deg = n_nodes * n_rel
    out_rows = -(-(n_nodes + 1) // (NS * 8)) * (NS * 8)
    opt = out_rows // NS
    trash_row = n_nodes

    mesh = plsc.VectorSubcoreMesh(core_axis_name="c", subcore_axis_name="s",
                                  num_cores=NC, num_subcores=NS)

    @functools.partial(
        pl.kernel, mesh=mesh,
        compiler_params=_SC_PARAMS,
        out_type=(
            jax.ShapeDtypeStruct((out_rows, L), jnp.float32),
            jax.ShapeDtypeStruct((out_rows, L), jnp.float32),
            jax.ShapeDtypeStruct((e,), jnp.float32),
            jax.ShapeDtypeStruct((e,), jnp.int32),
            jax.ShapeDtypeStruct((deg_flat,), jnp.float32),
            jax.ShapeDtypeStruct((deg_flat,), jnp.float32),
        ),
        scratch_types=[
            pltpu.VMEM_SHARED((deg_flat,), jnp.float32),
            pltpu.VMEM_SHARED((out_rows, L), jnp.float32),
            pltpu.VMEM((CH,), jnp.int32),
            pltpu.VMEM((CH,), jnp.int32),
            pltpu.VMEM((CH,), jnp.int32),
            pltpu.VMEM((CH,), jnp.int32),
            pltpu.VMEM((CH,), jnp.int32),
            pltpu.VMEM((CH,), jnp.float32),
            pltpu.VMEM((CH,), jnp.float32),
            pltpu.VMEM((CH,), jnp.float32),
            pltpu.VMEM((CH, L), jnp.float32),
            pltpu.VMEM((opt, L), jnp.float32),
            pltpu.VMEM((dpt,), jnp.float32),
            pltpu.SemaphoreType.DMA,
        ],
    )
    def k(ef, tp, ytab, outp0, outp1, normp, idxgp, deg0, deg1,
          sdeg, sout, src_v, dst_v, t_v, idxd_v, idxg_v, ones_v, dval_v,
          norm_v, yrows_v, stage_v, zbuf_v, sem):
        c = lax.axis_index("c")
        s = lax.axis_index("s")

        def init_ones(i, _):
            ones_v[pl.ds(i * L, L)] = jnp.full((L,), 1.0, jnp.float32)
            return 0
        lax.fori_loop(0, CH // L, init_ones, 0)
        _zero_flat(zbuf_v, dpt)
        _zero_rows(stage_v, opt)
        pltpu.sync_copy(zbuf_v, sdeg.at[pl.ds(s * dpt, dpt)])
        pltpu.sync_copy(stage_v, sout.at[pl.ds(s * opt, opt)])
        plsc.subcore_barrier()

        def p1_chunk(eb, n_real):
            if n_real == CH:
                pltpu.sync_copy(ef.at[pl.ds(e + eb, CH)], dst_v)
                pltpu.sync_copy(tp.at[pl.ds(eb, CH)], t_v)
            else:
                pltpu.sync_copy(ef.at[pl.ds(e + eb, n_real)],
                                dst_v.at[pl.ds(0, n_real)])
                pltpu.sync_copy(tp.at[pl.ds(eb, n_real)],
                                t_v.at[pl.ds(0, n_real)])
            fg = n_real // L

            def mk_idxd(g, _):
                sl = pl.ds(g * L, L)
                idxd_v[sl] = dst_v[sl] * n_rel + t_v[sl]
                return 0
            lax.fori_loop(0, fg, mk_idxd, 0)
            if n_real % L:
                sl = pl.ds(fg * L, L)
                m = lax.iota(jnp.int32, L) < (n_real % L)
                idxd_v[sl] = jnp.where(m, dst_v[sl] * n_rel + t_v[sl],
                                       trash_deg)
                fg += 1
            _fill_groups(idxd_v, fg, trash_deg)
            pltpu.sync_copy(ones_v, sdeg.at[idxd_v], add=True)

        nfull1 = qt1 // CH
        for kk in range(nfull1):
            p1_chunk(s * qt1 + kk * CH, CH)
        if qt1 % CH:
            p1_chunk(s * qt1 + nfull1 * CH, qt1 % CH)
        plsc.subcore_barrier()

        def deg_out(degref):
            pltpu.sync_copy(sdeg.at[pl.ds(s * dpt, dpt)], zbuf_v)
            pltpu.sync_copy(zbuf_v, degref.at[pl.ds(s * dpt, dpt)])

        @pl.when(c == 0)
        def _():
            deg_out(deg0)

        @pl.when(c == 1)
        def _():
            deg_out(deg1)
        plsc.subcore_barrier()

        def p3_chunk(eb, n_real):
            if n_real == CH:
                pltpu.sync_copy(ef.at[pl.ds(eb, CH)], src_v)
                pltpu.sync_copy(ef.at[pl.ds(e + eb, CH)], dst_v)
                pltpu.sync_copy(tp.at[pl.ds(eb, CH)], t_v)
            else:
                pltpu.sync_copy(ef.at[pl.ds(eb, n_real)],
                                src_v.at[pl.ds(0, n_real)])
                pltpu.sync_copy(ef.at[pl.ds(e + eb, n_real)],
                                dst_v.at[pl.ds(0, n_real)])
                pltpu.sync_copy(tp.at[pl.ds(eb, n_real)],
                                t_v.at[pl.ds(0, n_real)])
            fg = n_real // L

            def mk_idx(g, _):
                sl = pl.ds(g * L, L)
                tt = t_v[sl]
                idxg_v[sl] = src_v[sl] * n_rel + tt
                idxd_v[sl] = dst_v[sl] * n_rel + tt
                return 0
            lax.fori_loop(0, fg, mk_idx, 0)
            if n_real % L:
                sl = pl.ds(fg * L, L)
                m = lax.iota(jnp.int32, L) < (n_real % L)
                tt = t_v[sl]
                idxg_v[sl] = jnp.where(m, src_v[sl] * n_rel + tt, 0)
                idxd_v[sl] = jnp.where(m, dst_v[sl] * n_rel + tt, trash_deg)
                dst_v[sl] = jnp.where(m, dst_v[sl], trash_row)
                fg += 1
            _fill_groups(idxg_v, fg, 0)
            _fill_groups(idxd_v, fg, trash_deg)
            _fill_groups(dst_v, fg, trash_row)

            pltpu.sync_copy(idxg_v.at[pl.ds(0, n_real)],
                            idxgp.at[pl.ds(eb, n_real)])

            @pl.when(c == 0)
            def _():
                pltpu.async_copy(deg0.at[idxd_v], dval_v, sem).wait()

            @pl.when(c == 1)
            def _():
                pltpu.async_copy(deg1.at[idxd_v], dval_v, sem).wait()

            def mk_norm(g, _):
                sl = pl.ds(g * L, L)
                norm_v[sl] = 1.0 / jnp.maximum(dval_v[sl], 1.0)
                return 0
            lax.fori_loop(0, CH // L, mk_norm, 0)
            pltpu.sync_copy(norm_v.at[pl.ds(0, n_real)],
                            normp.at[pl.ds(eb, n_real)])

            pltpu.async_copy(ytab.at[idxg_v], yrows_v, sem).wait()
            _scale_rows(yrows_v, norm_v, CH)
            pltpu.sync_copy(yrows_v, sout.at[dst_v], add=True)

        nfull3 = qt3 // CH
        for kk in range(nfull3):
            p3_chunk(c * (e // NC) + s * qt3 + kk * CH, CH)
        if qt3 % CH:
            p3_chunk(c * (e // NC) + s * qt3 + nfull3 * CH, qt3 % CH)
        plsc.subcore_barrier()

        def part_out(oref):
            pltpu.sync_copy(sout.at[pl.ds(s * opt, opt)], stage_v)
            pltpu.sync_copy(stage_v, oref.at[pl.ds(s * opt, opt)])

        @pl.when(c == 0)
        def _():
            part_out(outp0)

        @pl.when(c == 1)
        def _():
            part_out(outp1)

    return k


def _make_sc_layer2(n_nodes, e):
    assert e % (NC * NS * 8) == 0
    qt = e // (NC * NS)
    out_rows = -(-(n_nodes + 1) // (NS * 8)) * (NS * 8)
    opt = out_rows // NS
    trash_row = n_nodes

    mesh = plsc.VectorSubcoreMesh(core_axis_name="c", subcore_axis_name="s",
                                  num_cores=NC, num_subcores=NS)

    @functools.partial(
        pl.kernel, mesh=mesh,
        compiler_params=_SC_PARAMS,
        out_type=(
            jax.ShapeDtypeStruct((out_rows, L), jnp.float32),
            jax.ShapeDtypeStruct((out_rows, L), jnp.float32),
        ),
        scratch_types=[
            pltpu.VMEM_SHARED((out_rows, L), jnp.float32),
            pltpu.VMEM((CH,), jnp.int32),
            pltpu.VMEM((CH,), jnp.int32),
            pltpu.VMEM((CH,), jnp.float32),
            pltpu.VMEM((CH, L), jnp.float32),
            pltpu.VMEM((opt, L), jnp.float32),
            pltpu.SemaphoreType.DMA,
        ],
    )
    def k(idxgp, ef, normp, ytab, outp0, outp1,
          sout, idxg_v, dst_v, norm_v, yrows_v, stage_v, sem):
        c = lax.axis_index("c")
        s = lax.axis_index("s")

        _zero_rows(stage_v, opt)
        pltpu.sync_copy(stage_v, sout.at[pl.ds(s * opt, opt)])
        plsc.subcore_barrier()

        def chunk(eb, n_real):
            if n_real == CH:
                pltpu.sync_copy(idxgp.at[pl.ds(eb, CH)], idxg_v)
                pltpu.sync_copy(ef.at[pl.ds(e + eb, CH)], dst_v)
                pltpu.sync_copy(normp.at[pl.ds(eb, CH)], norm_v)
            else:
                pltpu.sync_copy(idxgp.at[pl.ds(eb, n_real)],
                                idxg_v.at[pl.ds(0, n_real)])
                pltpu.sync_copy(ef.at[pl.ds(e + eb, n_real)],
                                dst_v.at[pl.ds(0, n_real)])
                pltpu.sync_copy(normp.at[pl.ds(eb, n_real)],
                                norm_v.at[pl.ds(0, n_real)])
            fg = n_real // L
            if n_real % L:
                sl = pl.ds(fg * L, L)
                m = lax.iota(jnp.int32, L) < (n_real % L)
                idxg_v[sl] = jnp.where(m, idxg_v[sl], 0)
                dst_v[sl] = jnp.where(m, dst_v[sl], trash_row)
                fg += 1
            _fill_groups(idxg_v, fg, 0)
            _fill_groups(dst_v, fg, trash_row)
            pltpu.async_copy(ytab.at[idxg_v], yrows_v, sem).wait()
            _scale_rows(yrows_v, norm_v, CH)
            pltpu.sync_copy(yrows_v, sout.at[dst_v], add=True)

        nfull = qt // CH
        for kk in range(nfull):
            chunk(c * (e // NC) + s * qt + kk * CH, CH)
        if qt % CH:
            chunk(c * (e // NC) + s * qt + nfull * CH, qt % CH)
        plsc.subcore_barrier()

        def part_out(oref):
            pltpu.sync_copy(sout.at[pl.ds(s * opt, opt)], stage_v)
            pltpu.sync_copy(stage_v, oref.at[pl.ds(s * opt, opt)])

        @pl.when(c == 0)
        def _():
            part_out(outp0)

        @pl.when(c == 1)
        def _():
            part_out(outp1)

    return k


def kernel(x, edge_index, edge_type, comp1, bases1, root1, bias1,
           comp2, bases2, root2, bias2):
    n_nodes, in_c = x.shape
    e = edge_index.shape[1]
    n_rel, n_bases = comp1.shape
    hid = bases1.shape[2]
    n_cls = bases2.shape[2]
    assert hid == L and n_cls == L

    w1 = _mm(comp1, bases1.reshape(n_bases, in_c * hid))
    w1 = w1.reshape(n_rel, in_c, hid).transpose(1, 0, 2).reshape(
        in_c, n_rel * hid)
    w2 = _mm(comp2, bases2.reshape(n_bases, hid * n_cls))
    w2 = w2.reshape(n_rel, hid, n_cls).transpose(1, 0, 2).reshape(
        hid, n_rel * n_cls)

    y1, xr1 = _mm_split(x, jnp.concatenate([w1, root1], axis=1),
                        n_rel * hid, 1000)
    y1tab = y1.reshape(n_nodes * n_rel, hid)

    sc1 = _make_sc_layer1(n_nodes, n_rel, e)
    eflat = edge_index.reshape(2 * e)
    p0, p1, normp, idxgp, _, _ = sc1(eflat, edge_type, y1tab)

    y2, xr2 = _relu_mm_split(
        p0, p1, xr1, bias1.reshape(1, hid),
        jnp.concatenate([w2, root2], axis=1), n_rel * n_cls, 1000)
    y2tab = y2.reshape(n_nodes * n_rel, n_cls)

    sc2 = _make_sc_layer2(n_nodes, e)
    q0, q1 = sc2(idxgp, eflat, normp, y2tab)

    return _final_logsoftmax(q0, q1, xr2, bias2.reshape(1, n_cls), 1000)
